# baseline (speedup 1.0000x reference)
import functools

import jax
import jax.numpy as jnp
from jax import lax
from jax.experimental import pallas as pl
from jax.experimental.pallas import tpu as pltpu
from jax.experimental.pallas import tpu_sc as plsc

D = 300
L = 16
NVREG = 19
DP = NVREG * L
B_SEG = 256
N_ROWS = 100000
NW = 32
CHUNK = 100
NCHUNK = N_ROWS // CHUNK
SEGP = 104
ACC_W = B_SEG * DP


def _sc_pool_body(x_hbm, seg_hbm, w_hbm, b_hbm, out_hbm,
                  xbuf, acc, segbuf, wbuf, bbuf):
    cid = lax.axis_index("c")
    sid = lax.axis_index("s")
    wid = sid * 2 + cid

    pltpu.sync_copy(w_hbm, wbuf)
    pltpu.sync_copy(b_hbm, bbuf)

    zero = jnp.zeros((L,), jnp.float32)

    def zrow(i, carry):
        acc[pl.ds(i * L, L)] = zero
        return carry

    lax.fori_loop(0, ACC_W // L, zrow, 0)

    wv = [wbuf[pl.ds(k * L, L)] for k in range(NVREG)]
    bv = bbuf[...]
    iota = lax.iota(jnp.int32, L)
    colidx = [iota + k * L for k in range(NVREG)]
    tmask = iota < (D - (NVREG - 1) * L)
    nc = jnp.where(wid < (NCHUNK - (NCHUNK // NW) * NW), NCHUNK // NW + 1,
                   NCHUNK // NW)

    def chunk_body(j, carry):
        g = wid + j * NW
        pltpu.sync_copy(x_hbm.at[pl.ds(g * (CHUNK * D), CHUNK * D)], xbuf)
        pltpu.sync_copy(seg_hbm.at[pl.ds(g * SEGP, SEGP)], segbuf)

        def row_body(r, rcarry):
            roff = jnp.zeros((L,), jnp.int32) + r * D
            t = zero
            xv = []
            for k in range(NVREG - 1):
                v = plsc.load_gather(xbuf, [roff + colidx[k]])
                xv.append(v)
                t = t + v * wv[k]
            vt = plsc.load_gather(xbuf, [roff + colidx[NVREG - 1]], mask=tmask)
            vt = jnp.where(tmask, vt, 0.0)
            xv.append(vt)
            t = t + vt * wv[NVREG - 1]
            s = jnp.sum(t)
            wgt = 1.0 / (1.0 + jnp.exp(-(s + bv)))
            segsplat = plsc.load_gather(segbuf, [jnp.zeros((L,), jnp.int32) + r])
            segoff = segsplat * DP
            for k in range(NVREG):
                plsc.addupdate_scatter(acc, [segoff + colidx[k]], wgt * xv[k])
            return rcarry

        lax.fori_loop(0, CHUNK, row_body, 0)
        return carry

    lax.fori_loop(0, nc, chunk_body, 0)

    pltpu.sync_copy(acc, out_hbm.at[pl.ds(wid * ACC_W, ACC_W)])


_sc_pool = functools.partial(
    pl.kernel,
    out_type=jax.ShapeDtypeStruct((NW * ACC_W,), jnp.float32),
    mesh=plsc.VectorSubcoreMesh(core_axis_name="c", subcore_axis_name="s",
                                num_cores=2, num_subcores=16),
    compiler_params=pltpu.CompilerParams(use_tc_tiling_on_sc=False,
                                         needs_layout_passes=False),
    scratch_types=[
        pltpu.VMEM((CHUNK * D,), jnp.float32),
        pltpu.VMEM((ACC_W,), jnp.float32),
        pltpu.VMEM((SEGP,), jnp.int32),
        pltpu.VMEM((DP,), jnp.float32),
        pltpu.VMEM((L,), jnp.float32),
    ],
)(_sc_pool_body)


def _reduce_parts(p_ref, o_ref):
    o_ref[...] = jnp.sum(p_ref[...], axis=0)[:, :D]


def kernel(x, segment_ids, batch_size, W, b):
    del batch_size
    seg = segment_ids.astype(jnp.int32).reshape(NCHUNK, CHUNK)
    seg = jnp.pad(seg, ((0, 0), (0, SEGP - CHUNK))).reshape(NCHUNK * SEGP)
    wp = jnp.concatenate([W.reshape(D), jnp.zeros((DP - D,), jnp.float32)])
    bs = jnp.broadcast_to(b.reshape(1).astype(jnp.float32), (L,))
    parts = _sc_pool(x.reshape(N_ROWS * D), seg, wp, bs)
    parts = parts.reshape(NW, B_SEG, DP)
    out = pl.pallas_call(
        _reduce_parts,
        out_shape=jax.ShapeDtypeStruct((B_SEG, D), jnp.float32),
    )(parts)
    return out


# speedup vs baseline: 1.7577x; 1.7577x over previous
import functools

import jax
import jax.numpy as jnp
from jax import lax
from jax.experimental import pallas as pl
from jax.experimental.pallas import tpu as pltpu
from jax.experimental.pallas import tpu_sc as plsc

D = 300
L = 16
NVREG = 19
DP = NVREG * L
B_SEG = 256
N_ROWS = 100000
NW = 32
CHUNK = 80
NCHUNK = N_ROWS // CHUNK
SEGP = 88
ACC_W = B_SEG * DP


def _sc_pool_body(x_hbm, seg_hbm, w_hbm, b_hbm, out_hbm,
                  xbuf, acc, segbuf, wbuf, bbuf):
    cid = lax.axis_index("c")
    sid = lax.axis_index("s")
    wid = sid * 2 + cid

    pltpu.sync_copy(w_hbm, wbuf)
    pltpu.sync_copy(b_hbm, bbuf)

    zero = jnp.zeros((L,), jnp.float32)

    def zrow(i, carry):
        acc[pl.ds(i * L, L)] = zero
        return carry

    lax.fori_loop(0, ACC_W // L, zrow, 0)

    wv = [wbuf[pl.ds(k * L, L)] for k in range(NVREG)]
    bv = bbuf[...]
    iota = lax.iota(jnp.int32, L)
    colidx = [iota + k * L for k in range(NVREG)]
    tmask = iota < (D - (NVREG - 1) * L)
    nc = jnp.where(wid < (NCHUNK - (NCHUNK // NW) * NW), NCHUNK // NW + 1,
                   NCHUNK // NW)

    def chunk_body(j, carry):
        g = wid + j * NW
        pltpu.sync_copy(x_hbm.at[pl.ds(g * CHUNK, CHUNK), :], xbuf)
        pltpu.sync_copy(seg_hbm.at[pl.ds(g * SEGP, SEGP)], segbuf)

        def row_body(r, rcarry):
            rsplat = jnp.zeros((L,), jnp.int32) + r
            t = zero
            xv = []
            for k in range(NVREG - 1):
                v = plsc.load_gather(xbuf, [rsplat, colidx[k]])
                xv.append(v)
                t = t + v * wv[k]
            vt = plsc.load_gather(xbuf, [rsplat, colidx[NVREG - 1]], mask=tmask)
            vt = jnp.where(tmask, vt, 0.0)
            xv.append(vt)
            t = t + vt * wv[NVREG - 1]
            s = jnp.sum(t)
            wgt = 1.0 / (1.0 + jnp.exp(-(s + bv)))
            segsplat = plsc.load_gather(segbuf, [jnp.zeros((L,), jnp.int32) + r])
            segoff = segsplat * DP
            for k in range(NVREG):
                plsc.addupdate_scatter(acc, [segoff + colidx[k]], wgt * xv[k])
            return rcarry

        lax.fori_loop(0, CHUNK, row_body, 0)
        return carry

    lax.fori_loop(0, nc, chunk_body, 0)

    pltpu.sync_copy(acc, out_hbm.at[pl.ds(wid * ACC_W, ACC_W)])


_sc_pool = functools.partial(
    pl.kernel,
    out_type=jax.ShapeDtypeStruct((NW * ACC_W,), jnp.float32),
    mesh=plsc.VectorSubcoreMesh(core_axis_name="c", subcore_axis_name="s",
                                num_cores=2, num_subcores=16),
    compiler_params=pltpu.CompilerParams(use_tc_tiling_on_sc=True,
                                         needs_layout_passes=False),
    scratch_types=[
        pltpu.VMEM((CHUNK, D), jnp.float32),
        pltpu.VMEM((ACC_W,), jnp.float32),
        pltpu.VMEM((SEGP,), jnp.int32),
        pltpu.VMEM((DP,), jnp.float32),
        pltpu.VMEM((L,), jnp.float32),
    ],
)(_sc_pool_body)


def _reduce_parts(p_ref, o_ref):
    o_ref[...] = jnp.sum(p_ref[...], axis=0)[:, :D]


def kernel(x, segment_ids, batch_size, W, b):
    del batch_size
    seg = segment_ids.astype(jnp.int32).reshape(NCHUNK, CHUNK)
    seg = jnp.pad(seg, ((0, 0), (0, SEGP - CHUNK))).reshape(NCHUNK * SEGP)
    wp = jnp.concatenate([W.reshape(D), jnp.zeros((DP - D,), jnp.float32)])
    bs = jnp.broadcast_to(b.reshape(1).astype(jnp.float32), (L,))
    parts = _sc_pool(x, seg, wp, bs)
    parts = parts.reshape(NW, B_SEG, DP)
    out = pl.pallas_call(
        _reduce_parts,
        out_shape=jax.ShapeDtypeStruct((B_SEG, D), jnp.float32),
    )(parts)
    return out


# speedup vs baseline: 2.6593x; 1.5130x over previous
import functools

import jax
import jax.numpy as jnp
from jax import lax
from jax.experimental import pallas as pl
from jax.experimental.pallas import tpu as pltpu
from jax.experimental.pallas import tpu_sc as plsc

D = 300
L = 16
NVREG = 19
DP = NVREG * L
B_SEG = 256
N_ROWS = 100000
NW = 32
CHUNK = 40
NCHUNK = N_ROWS // CHUNK
NC_LO = NCHUNK // NW
NC_REM = NCHUNK - NC_LO * NW
NC_MAX = NC_LO + 1
SEG_LEN = NC_MAX * CHUNK
ACC_W = B_SEG * DP
RUNROLL = 4


def _sc_pool_body(x_hbm, seg_hbm, w_hbm, b_hbm, out_hbm,
                  xbuf0, xbuf1, acc, segbuf, wbuf, bbuf, sem0, sem1):
    cid = lax.axis_index("c")
    sid = lax.axis_index("s")
    wid = sid * 2 + cid
    nc = NC_LO + (wid < NC_REM).astype(jnp.int32)
    gbase = NC_LO * wid + jnp.minimum(wid, NC_REM)

    pltpu.sync_copy(w_hbm, wbuf)
    pltpu.sync_copy(b_hbm, bbuf)
    pltpu.sync_copy(seg_hbm.at[pl.ds(gbase * CHUNK, SEG_LEN)], segbuf)

    zero = jnp.zeros((L,), jnp.float32)
    zero_i = jnp.zeros((L,), jnp.int32)

    def zrow(i, carry):
        for u in range(8):
            acc[pl.ds((i * 8 + u) * L, L)] = zero
        return carry

    lax.fori_loop(0, ACC_W // (L * 8), zrow, 0)

    wv = [wbuf[pl.ds(k * L, L)] for k in range(NVREG)]
    bv = bbuf[...]
    iota = lax.iota(jnp.int32, L)
    colidx = [iota + k * L for k in range(NVREG)]
    tmask = iota < (D - (NVREG - 1) * L)
    perms = [jnp.bitwise_xor(iota, 1 << s) for s in range(4)]

    def xcopy(g, buf, sem):
        return pltpu.async_copy(x_hbm.at[pl.ds(g * CHUNK, CHUNK), :], buf, sem)

    def xwait(buf, sem):
        pltpu.make_async_copy(x_hbm.at[pl.ds(0, CHUNK), :], buf, sem).wait()

    def process_row(buf, j, r):
        rsplat = zero_i + r
        ts = [zero, zero, zero, zero]
        xv = []
        for k in range(NVREG - 1):
            v = plsc.load_gather(buf, [rsplat, colidx[k]])
            xv.append(v)
            ts[k & 3] = ts[k & 3] + v * wv[k]
        vt = plsc.load_gather(buf, [rsplat, colidx[NVREG - 1]], mask=tmask)
        vt = jnp.where(tmask, vt, 0.0)
        xv.append(vt)
        ts[(NVREG - 1) & 3] = ts[(NVREG - 1) & 3] + vt * wv[NVREG - 1]
        t = (ts[0] + ts[1]) + (ts[2] + ts[3])
        for p in perms:
            t = t + t.at[p].get(mode="promise_in_bounds",
                                unique_indices=True)
        wgt = 1.0 / (1.0 + jnp.exp(-(t + bv)))
        segsplat = plsc.load_gather(segbuf, [zero_i + (j * CHUNK + r)])
        segoff = segsplat * DP
        for k in range(NVREG):
            plsc.addupdate_scatter(acc, [segoff + colidx[k]], wgt * xv[k])

    def do_chunk(j, buf, sem, obuf, osem):
        @pl.when(j < nc)
        def _():
            xwait(buf, sem)

            @pl.when(j + 1 < nc)
            def _():
                xcopy(gbase + j + 1, obuf, osem)

            def rows(i, carry):
                for u in range(RUNROLL):
                    process_row(buf, j, i * RUNROLL + u)
                return carry

            lax.fori_loop(0, CHUNK // RUNROLL, rows, 0)

    xcopy(gbase, xbuf0, sem0)

    def pair(jp, carry):
        do_chunk(jp * 2, xbuf0, sem0, xbuf1, sem1)
        do_chunk(jp * 2 + 1, xbuf1, sem1, xbuf0, sem0)
        return carry

    lax.fori_loop(0, (NC_MAX + 1) // 2, pair, 0)

    pltpu.sync_copy(acc, out_hbm.at[pl.ds(wid * ACC_W, ACC_W)])


_sc_pool = functools.partial(
    pl.kernel,
    out_type=jax.ShapeDtypeStruct((NW * ACC_W,), jnp.float32),
    mesh=plsc.VectorSubcoreMesh(core_axis_name="c", subcore_axis_name="s",
                                num_cores=2, num_subcores=16),
    compiler_params=pltpu.CompilerParams(use_tc_tiling_on_sc=True,
                                         needs_layout_passes=False),
    scratch_types=[
        pltpu.VMEM((CHUNK, D), jnp.float32),
        pltpu.VMEM((CHUNK, D), jnp.float32),
        pltpu.VMEM((ACC_W,), jnp.float32),
        pltpu.VMEM((SEG_LEN,), jnp.int32),
        pltpu.VMEM((DP,), jnp.float32),
        pltpu.VMEM((L,), jnp.float32),
        pltpu.SemaphoreType.DMA,
        pltpu.SemaphoreType.DMA,
    ],
)(_sc_pool_body)


def _reduce_parts(p_ref, o_ref):
    o_ref[...] = jnp.sum(p_ref[...], axis=0)[:, :D]


def kernel(x, segment_ids, batch_size, W, b):
    del batch_size
    seg = jnp.pad(segment_ids.astype(jnp.int32), (0, SEG_LEN))
    wp = jnp.concatenate([W.reshape(D), jnp.zeros((DP - D,), jnp.float32)])
    bs = jnp.broadcast_to(b.reshape(1).astype(jnp.float32), (L,))
    parts = _sc_pool(x, seg, wp, bs)
    parts = parts.reshape(NW, B_SEG, DP)
    out = pl.pallas_call(
        _reduce_parts,
        out_shape=jax.ShapeDtypeStruct((B_SEG, D), jnp.float32),
    )(parts)
    return out


# speedup vs baseline: 2.9171x; 1.0970x over previous
import functools

import jax
import jax.numpy as jnp
from jax import lax
from jax.experimental import pallas as pl
from jax.experimental.pallas import tpu as pltpu
from jax.experimental.pallas import tpu_sc as plsc

D = 300
L = 16
NVREG = 19
DP = NVREG * L
B_SEG = 256
N_ROWS = 100000
NW = 32
CHUNK = 40
NCHUNK = N_ROWS // CHUNK
NC_LO = NCHUNK // NW
NC_REM = NCHUNK - NC_LO * NW
NC_MAX = NC_LO + 1
SEG_LEN = NC_MAX * CHUNK
ACC_W = B_SEG * DP
RUNROLL = 4


def _sc_pool_body(x_hbm, seg_hbm, w_hbm, b_hbm, out_hbm,
                  xbuf0, xbuf1, acc, segbuf, wbuf, bbuf, sem0, sem1):
    cid = lax.axis_index("c")
    sid = lax.axis_index("s")
    wid = sid * 2 + cid
    nc = NC_LO + (wid < NC_REM).astype(jnp.int32)
    gbase = NC_LO * wid + jnp.minimum(wid, NC_REM)

    pltpu.sync_copy(w_hbm, wbuf)
    pltpu.sync_copy(b_hbm, bbuf)
    pltpu.sync_copy(seg_hbm.at[pl.ds(gbase * CHUNK, SEG_LEN)], segbuf)

    zero = jnp.zeros((L,), jnp.float32)
    zero_i = jnp.zeros((L,), jnp.int32)

    def zrow(i, carry):
        for u in range(8):
            acc[pl.ds((i * 8 + u) * L, L)] = zero
        return carry

    lax.fori_loop(0, ACC_W // (L * 8), zrow, 0)

    wv = [wbuf[pl.ds(k * L, L)] for k in range(NVREG)]
    bv = bbuf[...]
    iota = lax.iota(jnp.int32, L)
    colidx = [iota + k * L for k in range(NVREG)]
    tmask = iota < (D - (NVREG - 1) * L)
    perms = [jnp.bitwise_xor(iota, 1 << s) for s in range(4)]

    def xcopy(g, buf, sem):
        return pltpu.async_copy(x_hbm.at[pl.ds(g * CHUNK, CHUNK), :], buf, sem)

    def xwait(buf, sem):
        pltpu.make_async_copy(x_hbm.at[pl.ds(0, CHUNK), :], buf, sem).wait()

    def process_row(buf, j, r):
        rsplat = zero_i + r
        ts = [zero, zero, zero, zero]
        xv = []
        for k in range(NVREG - 1):
            v = buf[r, pl.ds(k * L, L)]
            xv.append(v)
            ts[k & 3] = ts[k & 3] + v * wv[k]
        vt = plsc.load_gather(buf, [rsplat, colidx[NVREG - 1]], mask=tmask)
        vt = jnp.where(tmask, vt, 0.0)
        xv.append(vt)
        ts[(NVREG - 1) & 3] = ts[(NVREG - 1) & 3] + vt * wv[NVREG - 1]
        t = (ts[0] + ts[1]) + (ts[2] + ts[3])
        for p in perms:
            t = t + t.at[p].get(mode="promise_in_bounds",
                                unique_indices=True)
        wgt = 1.0 / (1.0 + jnp.exp(-(t + bv)))
        segsplat = plsc.load_gather(segbuf, [zero_i + (j * CHUNK + r)])
        segoff = segsplat * DP
        for k in range(NVREG):
            plsc.addupdate_scatter(acc, [segoff + colidx[k]], wgt * xv[k])

    def do_chunk(j, buf, sem, obuf, osem):
        @pl.when(j < nc)
        def _():
            xwait(buf, sem)

            @pl.when(j + 1 < nc)
            def _():
                xcopy(gbase + j + 1, obuf, osem)

            def rows(i, carry):
                for u in range(RUNROLL):
                    process_row(buf, j, i * RUNROLL + u)
                return carry

            lax.fori_loop(0, CHUNK // RUNROLL, rows, 0)

    xcopy(gbase, xbuf0, sem0)

    def pair(jp, carry):
        do_chunk(jp * 2, xbuf0, sem0, xbuf1, sem1)
        do_chunk(jp * 2 + 1, xbuf1, sem1, xbuf0, sem0)
        return carry

    lax.fori_loop(0, (NC_MAX + 1) // 2, pair, 0)

    pltpu.sync_copy(acc, out_hbm.at[pl.ds(wid * ACC_W, ACC_W)])


_sc_pool = functools.partial(
    pl.kernel,
    out_type=jax.ShapeDtypeStruct((NW * ACC_W,), jnp.float32),
    mesh=plsc.VectorSubcoreMesh(core_axis_name="c", subcore_axis_name="s",
                                num_cores=2, num_subcores=16),
    compiler_params=pltpu.CompilerParams(use_tc_tiling_on_sc=True,
                                         needs_layout_passes=False),
    scratch_types=[
        pltpu.VMEM((CHUNK, D), jnp.float32),
        pltpu.VMEM((CHUNK, D), jnp.float32),
        pltpu.VMEM((ACC_W,), jnp.float32),
        pltpu.VMEM((SEG_LEN,), jnp.int32),
        pltpu.VMEM((DP,), jnp.float32),
        pltpu.VMEM((L,), jnp.float32),
        pltpu.SemaphoreType.DMA,
        pltpu.SemaphoreType.DMA,
    ],
)(_sc_pool_body)


def _reduce_parts(p_ref, o_ref):
    o_ref[...] = jnp.sum(p_ref[...], axis=0)[:, :D]


def kernel(x, segment_ids, batch_size, W, b):
    del batch_size
    seg = jnp.pad(segment_ids.astype(jnp.int32), (0, SEG_LEN))
    wp = jnp.concatenate([W.reshape(D), jnp.zeros((DP - D,), jnp.float32)])
    bs = jnp.broadcast_to(b.reshape(1).astype(jnp.float32), (L,))
    parts = _sc_pool(x, seg, wp, bs)
    parts = parts.reshape(NW, B_SEG, DP)
    out = pl.pallas_call(
        _reduce_parts,
        out_shape=jax.ShapeDtypeStruct((B_SEG, D), jnp.float32),
    )(parts)
    return out
